# Initial kernel scaffold; baseline (speedup 1.0000x reference)
import jax
import jax.numpy as jnp
from jax.experimental import pallas as pl


def kernel(embeddings, volatility, risk_factors, Wr, br, W1, b1, W2, b2):
    raise NotImplementedError("write your pallas kernel here")


# speedup vs baseline: 1.4337x; 1.4337x over previous
import functools

import jax
import jax.numpy as jnp
from jax.experimental import pallas as pl
from jax.experimental.pallas import tpu as pltpu

_B, _S, _D = 2, 2048, 1024
_E, _TOPK, _DFF = 8, 2, 2048
_T = _B * _S
_TM = 256
_G = _T * _TOPK
_NT = _G // _TM
_WS = _NT + _E - 1
_LANES = 128


def _router_body(x_ref, vr_ref, wr_ref, wsm_ref, br_ref, idx_ref, gate_ref):
    logits = jnp.dot(x_ref[...], wr_ref[...], preferred_element_type=jnp.float32)
    logits = logits + jnp.dot(vr_ref[...], wsm_ref[...],
                              preferred_element_type=jnp.float32)
    logits = logits + br_ref[...]
    lane = jax.lax.broadcasted_iota(jnp.int32, logits.shape, 1)
    logits = jnp.where(lane < _E, logits, -1e30)
    m1 = jnp.max(logits, axis=1, keepdims=True)
    i1 = jnp.min(jnp.where(logits == m1, lane, _LANES), axis=1, keepdims=True)
    rest = jnp.where(lane == i1, -1e30, logits)
    m2 = jnp.max(rest, axis=1, keepdims=True)
    i2 = jnp.min(jnp.where(rest == m2, lane, _LANES), axis=1, keepdims=True)
    w1 = 1.0 / (1.0 + jnp.exp(m2 - m1))
    w2 = 1.0 - w1
    idx_ref[...] = jnp.where(lane == 0, i1, jnp.where(lane == 1, i2, 0))
    gate_ref[...] = jnp.where(lane == 0, w1, jnp.where(lane == 1, w2, 0.0))


def _run_router(x, vr, wr_pad, wsm, br2):
    grid = (_T // _TM,)
    return pl.pallas_call(
        _router_body,
        grid=grid,
        in_specs=[
            pl.BlockSpec((_TM, _D), lambda i: (i, 0)),
            pl.BlockSpec((_TM, _LANES), lambda i: (i, 0)),
            pl.BlockSpec((_D, _LANES), lambda i: (0, 0)),
            pl.BlockSpec((_LANES, _LANES), lambda i: (0, 0)),
            pl.BlockSpec((1, _LANES), lambda i: (0, 0)),
        ],
        out_specs=[
            pl.BlockSpec((_TM, _LANES), lambda i: (i, 0)),
            pl.BlockSpec((_TM, _LANES), lambda i: (i, 0)),
        ],
        out_shape=[
            jax.ShapeDtypeStruct((_T, _LANES), jnp.int32),
            jax.ShapeDtypeStruct((_T, _LANES), jnp.float32),
        ],
    )(x, vr, wr_pad, wsm, br2)


def _ggemm_body(tid_r, eid_r, rs_r, re_r, init_r,
                x_ref, w1_ref, b1_ref, w2_ref, b2_ref, g_ref, y_ref):
    w = pl.program_id(0)
    h = jnp.dot(x_ref[...], w1_ref[0], preferred_element_type=jnp.float32)
    h = jax.nn.gelu(h + b1_ref[0])
    y = jnp.dot(h.astype(jnp.bfloat16), w2_ref[0],
                preferred_element_type=jnp.float32)
    y = (y + b2_ref[0]) * g_ref[...]
    rows = tid_r[w] * _TM + jax.lax.broadcasted_iota(jnp.int32, (_TM, 1), 0)
    contrib = jnp.where((rows >= rs_r[w]) & (rows < re_r[w]), y, 0.0)

    @pl.when(init_r[w] != 0)
    def _():
        y_ref[...] = contrib

    @pl.when(init_r[w] == 0)
    def _():
        y_ref[...] = y_ref[...] + contrib


def _run_ggemm(meta, xs, w1, b1, w2, b2, gs):
    tid, eid, rs, re, init = meta
    grid_spec = pltpu.PrefetchScalarGridSpec(
        num_scalar_prefetch=5,
        grid=(_WS,),
        in_specs=[
            pl.BlockSpec((_TM, _D), lambda w, tid, eid, rs, re, init: (tid[w], 0)),
            pl.BlockSpec((1, _D, _DFF), lambda w, tid, eid, rs, re, init: (eid[w], 0, 0)),
            pl.BlockSpec((1, 1, _DFF), lambda w, tid, eid, rs, re, init: (eid[w], 0, 0)),
            pl.BlockSpec((1, _DFF, _D), lambda w, tid, eid, rs, re, init: (eid[w], 0, 0)),
            pl.BlockSpec((1, 1, _D), lambda w, tid, eid, rs, re, init: (eid[w], 0, 0)),
            pl.BlockSpec((_TM, 1), lambda w, tid, eid, rs, re, init: (tid[w], 0)),
        ],
        out_specs=pl.BlockSpec((_TM, _D), lambda w, tid, eid, rs, re, init: (tid[w], 0)),
    )
    return pl.pallas_call(
        _ggemm_body,
        grid_spec=grid_spec,
        out_shape=jax.ShapeDtypeStruct((_G, _D), jnp.float32),
        compiler_params=pltpu.CompilerParams(
            dimension_semantics=("arbitrary",)),
    )(tid, eid, rs, re, init, xs, w1, b1, w2, b2, gs)


def _make_metadata(counts):
    off = jnp.concatenate([jnp.zeros((1,), jnp.int32),
                           jnp.cumsum(counts).astype(jnp.int32)])
    first_tile = off[:_E] // _TM
    last_tile = jnp.where(counts > 0, (off[1:] - 1) // _TM, first_tile - 1)
    num_items = last_tile - first_tile + 1
    cum = jnp.cumsum(num_items)
    total = cum[-1]
    w = jnp.arange(_WS, dtype=jnp.int32)
    eid = jnp.searchsorted(cum, w, side="right").astype(jnp.int32)
    valid = w < total
    eidc = jnp.clip(eid, 0, _E - 1)
    item_start = cum[eidc] - num_items[eidc]
    tid = first_tile[eidc] + (w - item_start)
    rs = jnp.maximum(off[eidc], tid * _TM)
    re = jnp.minimum(off[eidc + 1], (tid + 1) * _TM)
    init = ((rs == tid * _TM) & valid).astype(jnp.int32)
    tid = jnp.where(valid, tid, _NT - 1).astype(jnp.int32)
    rs = jnp.where(valid, rs, _G).astype(jnp.int32)
    re = jnp.where(valid, re, _G).astype(jnp.int32)
    eid = jnp.where(valid, eidc, _E - 1).astype(jnp.int32)
    return tid, eid, rs, re, init


def kernel(embeddings, volatility, risk_factors, Wr, br, W1, b1, W2, b2):
    x = embeddings.reshape(_T, _D)
    vol = volatility.reshape(_T, 1)
    risk = risk_factors.reshape(_T, 1)

    vr = jnp.zeros((_T, _LANES), jnp.float32).at[:, 0:1].set(vol).at[:, 1:2].set(risk)
    wr_pad = jnp.zeros((_D, _LANES), jnp.float32).at[:, :_E].set(Wr[:_D])
    wsm = jnp.zeros((_LANES, _LANES), jnp.float32).at[0, :_E].set(Wr[_D]).at[1, :_E].set(Wr[_D + 1])
    br2 = jnp.zeros((1, _LANES), jnp.float32).at[0, :_E].set(br)

    idx_out, gate_out = _run_router(x, vr, wr_pad, wsm, br2)
    flat_e = idx_out[:, :_TOPK].reshape(-1)
    flat_w = gate_out[:, :_TOPK].reshape(-1)
    flat_t = jnp.arange(_T, dtype=jnp.int32).repeat(_TOPK)

    perm = jnp.argsort(flat_e, stable=True)
    counts = jnp.bincount(flat_e, length=_E).astype(jnp.int32)
    meta = _make_metadata(counts)

    xs = jnp.take(x.astype(jnp.bfloat16), flat_t[perm], axis=0)
    gs = flat_w[perm].reshape(_G, 1)

    ys = _run_ggemm(meta, xs, W1.astype(jnp.bfloat16), b1.reshape(_E, 1, _DFF),
                    W2.astype(jnp.bfloat16), b2.reshape(_E, 1, _D), gs)

    dest = jnp.zeros((_G,), jnp.int32).at[perm].set(jnp.arange(_G, dtype=jnp.int32))
    out = ys[dest[0::2]] + ys[dest[1::2]]
    return out.reshape(_B, _S, _D)
